# baseline (speedup 1.0000x reference)
import functools

import jax
import jax.numpy as jnp
from jax import lax
from jax.experimental import pallas as pl
from jax.experimental.pallas import tpu as pltpu
from jax.experimental.pallas import tpu_sc as plsc

_B = 16384
_D = 64
_N = 1000000
_NC = 2
_NS = 16
_NW = _NC * _NS
_BPW = _B // _NW
_L = 16
_C = 128
_NCHUNK = _BPW // _C
_GPC = _C // _L
_W = 128


@functools.partial(
    pl.kernel,
    mesh=plsc.VectorSubcoreMesh(core_axis_name="c", subcore_axis_name="s"),
    out_type=jax.ShapeDtypeStruct((_B,), jnp.float32),
    compiler_params=pltpu.CompilerParams(
        needs_layout_passes=False, use_tc_tiling_on_sc=True),
    scratch_types=[
        pltpu.VMEM((_BPW,), jnp.int32),
        pltpu.VMEM((_BPW,), jnp.int32),
        pltpu.VMEM((_BPW,), jnp.int32),
        pltpu.VMEM((_BPW,), jnp.int32),
        pltpu.VMEM((_BPW,), jnp.int32),
        pltpu.VMEM((_BPW,), jnp.int32),
        pltpu.VMEM((_C, _W), jnp.float32),
        pltpu.VMEM((_C, _W), jnp.float32),
        pltpu.VMEM((_C, _W), jnp.float32),
        pltpu.VMEM((_C, _W), jnp.float32),
        pltpu.VMEM((_L,), jnp.float32),
        pltpu.VMEM((_BPW,), jnp.float32),
        pltpu.SemaphoreType.DMA,
    ],
)
def _mf_score(uid_hbm, iid_hbm, uw_hbm, iw_hbm, ub_hbm, ib_hbm,
              gb_hbm, out_hbm,
              uid_v, iid_v, ur_v, ir_v, ubr_v, ibr_v,
              ue_v, ie_v, ub_v, ib_v, gb_v, out_v, sem):
    wid = lax.axis_index("s") * _NC + lax.axis_index("c")
    base = wid * _BPW

    pltpu.sync_copy(uid_hbm.at[pl.ds(base, _BPW)], uid_v)
    pltpu.sync_copy(iid_hbm.at[pl.ds(base, _BPW)], iid_v)
    pltpu.sync_copy(gb_hbm, gb_v)

    def mkidx(g, carry):
        off = g * _L
        u16 = uid_v[pl.ds(off, _L)]
        i16 = iid_v[pl.ds(off, _L)]
        ur_v[pl.ds(off, _L)] = u16 >> 1
        ir_v[pl.ds(off, _L)] = i16 >> 1
        ubr_v[pl.ds(off, _L)] = u16 >> 7
        ibr_v[pl.ds(off, _L)] = i16 >> 7
        return carry

    lax.fori_loop(0, _BPW // _L, mkidx, 0)

    gb = gb_v[pl.ds(0, _L)]
    lane = lax.iota(jnp.int32, _L)

    for c in range(_NCHUNK):
        coff = c * _C
        c0 = pltpu.async_copy(uw_hbm.at[ur_v.at[pl.ds(coff, _C)]], ue_v, sem)
        c1 = pltpu.async_copy(iw_hbm.at[ir_v.at[pl.ds(coff, _C)]], ie_v, sem)
        c2 = pltpu.async_copy(ub_hbm.at[ubr_v.at[pl.ds(coff, _C)]], ub_v, sem)
        c3 = pltpu.async_copy(ib_hbm.at[ibr_v.at[pl.ds(coff, _C)]], ib_v, sem)
        c0.wait()
        c1.wait()
        c2.wait()
        c3.wait()

        def group(g, carry):
            goff = g * _L
            rows = goff + lane
            u16 = uid_v[pl.ds(coff + goff, _L)]
            i16 = iid_v[pl.ds(coff + goff, _L)]
            ucol = (u16 & 1) * _D
            icol = (i16 & 1) * _D
            acc = (plsc.load_gather(ub_v, [rows, u16 & 127])
                   + plsc.load_gather(ib_v, [rows, i16 & 127]) + gb)
            for d in range(_D):
                u = plsc.load_gather(ue_v, [rows, ucol + d])
                it = plsc.load_gather(ie_v, [rows, icol + d])
                acc = acc + u * it
            out_v[pl.ds(coff + goff, _L)] = acc
            return carry

        lax.fori_loop(0, _GPC, group, 0)

    pltpu.sync_copy(out_v, out_hbm.at[pl.ds(base, _BPW)])


def kernel(user_ids, item_ids, user_emb, item_emb, user_bias, item_bias,
           global_bias):
    uid = user_ids.astype(jnp.int32)
    iid = item_ids.astype(jnp.int32)
    uw = user_emb.reshape(_N // 2, _W)
    iw = item_emb.reshape(_N // 2, _W)
    pad = (-_N) % _W
    ub = jnp.pad(user_bias, ((0, pad), (0, 0))).reshape((_N + pad) // _W, _W)
    ib = jnp.pad(item_bias, ((0, pad), (0, 0))).reshape((_N + pad) // _W, _W)
    gb = jnp.broadcast_to(global_bias.reshape(-1)[:1], (_L,))
    return _mf_score(uid, iid, uw, iw, ub, ib, gb)


# speedup vs baseline: 2.2195x; 2.2195x over previous
import functools

import jax
import jax.numpy as jnp
from jax import lax
from jax.experimental import pallas as pl
from jax.experimental.pallas import tpu as pltpu
from jax.experimental.pallas import tpu_sc as plsc

_B = 16384
_D = 64
_N = 1000000
_NC = 2
_NS = 16
_NW = _NC * _NS
_BPW = _B // _NW
_L = 16
_G = _BPW // _L
_W = 128


@functools.partial(
    pl.kernel,
    mesh=plsc.VectorSubcoreMesh(core_axis_name="c", subcore_axis_name="s"),
    out_type=jax.ShapeDtypeStruct((_B,), jnp.float32),
    compiler_params=pltpu.CompilerParams(
        needs_layout_passes=False, use_tc_tiling_on_sc=True),
    scratch_types=[
        pltpu.VMEM((_BPW,), jnp.int32),
        pltpu.VMEM((_BPW,), jnp.int32),
        pltpu.VMEM((_D, _W), jnp.float32),
        pltpu.VMEM((_D, _W), jnp.float32),
        pltpu.VMEM((_D, _W), jnp.float32),
        pltpu.VMEM((_D, _W), jnp.float32),
        pltpu.VMEM((1, _W), jnp.float32),
        pltpu.VMEM((1, _W), jnp.float32),
        pltpu.VMEM((1, _W), jnp.float32),
        pltpu.VMEM((1, _W), jnp.float32),
        pltpu.VMEM((_L,), jnp.float32),
        pltpu.VMEM((_BPW,), jnp.float32),
        pltpu.SemaphoreType.DMA,
        pltpu.SemaphoreType.DMA,
    ],
)
def _mf_score(uid_hbm, iid_hbm, uT_hbm, iT_hbm, ubT_hbm, ibT_hbm,
              gb_hbm, out_hbm,
              uid_v, iid_v,
              ue0_v, ue1_v, ie0_v, ie1_v, ub0_v, ub1_v, ib0_v, ib1_v,
              gb_v, out_v, sem0, sem1):
    wid = lax.axis_index("s") * _NC + lax.axis_index("c")
    base = wid * _BPW

    pltpu.sync_copy(uid_hbm.at[pl.ds(base, _BPW)], uid_v)
    pltpu.sync_copy(iid_hbm.at[pl.ds(base, _BPW)], iid_v)
    pltpu.sync_copy(gb_hbm, gb_v)

    ue = (ue0_v, ue1_v)
    ie = (ie0_v, ie1_v)
    ub = (ub0_v, ub1_v)
    ib = (ib0_v, ib1_v)
    sems = (sem0, sem1)

    def fire(uscalar, iscalar, p):
        uoff = pl.multiple_of((uscalar >> 7) * _W, _W)
        ioff = pl.multiple_of((iscalar >> 7) * _W, _W)
        pltpu.async_copy(uT_hbm.at[:, pl.ds(uoff, _W)], ue[p], sems[p])
        pltpu.async_copy(iT_hbm.at[:, pl.ds(ioff, _W)], ie[p], sems[p])
        pltpu.async_copy(ubT_hbm.at[:, pl.ds(uoff, _W)], ub[p], sems[p])
        pltpu.async_copy(ibT_hbm.at[:, pl.ds(ioff, _W)], ib[p], sems[p])

    def drain(p):
        pltpu.make_async_copy(uT_hbm.at[:, pl.ds(0, _W)], ue[p], sems[p]).wait()
        pltpu.make_async_copy(iT_hbm.at[:, pl.ds(0, _W)], ie[p], sems[p]).wait()
        pltpu.make_async_copy(ubT_hbm.at[:, pl.ds(0, _W)], ub[p], sems[p]).wait()
        pltpu.make_async_copy(ibT_hbm.at[:, pl.ds(0, _W)], ib[p], sems[p]).wait()

    u16p = uid_v[pl.ds(0, _L)]
    i16p = iid_v[pl.ds(0, _L)]
    fire(u16p[0], i16p[0], 0)
    fire(u16p[1], i16p[1], 1)

    gb0 = gb_v[pl.ds(0, _L)][0]
    lane = lax.iota(jnp.int32, _L)
    zero16 = jnp.zeros((_L,), jnp.int32)

    def group(g, acc16):
        u16 = uid_v[pl.ds(g * _L, _L)]
        i16 = iid_v[pl.ds(g * _L, _L)]
        gnext = jnp.minimum(g + 1, _G - 1) * _L
        un16 = uid_v[pl.ds(gnext, _L)]
        in16 = iid_v[pl.ds(gnext, _L)]

        for j in range(_L):
            p = j & 1
            drain(p)
            cu = jnp.full((_L,), u16[j] & (_W - 1), jnp.int32)
            ci = jnp.full((_L,), i16[j] & (_W - 1), jnp.int32)
            s = jnp.zeros((_L,), jnp.float32)
            for q in range(_D // _L):
                rows = q * _L + lane
                u = plsc.load_gather(ue[p], [rows, cu])
                it = plsc.load_gather(ie[p], [rows, ci])
                s = s + u * it
            bias = (plsc.load_gather(ub[p], [zero16, cu])
                    + plsc.load_gather(ib[p], [zero16, ci]))

            if j < _L - 2:
                fire(u16[j + 2], i16[j + 2], p)
            else:
                @pl.when(g < _G - 1)
                def _(j=j, p=p):
                    fire(un16[j + 2 - _L], in16[j + 2 - _L], p)

            total = jnp.sum(s) + bias[0] + gb0
            acc16 = jnp.where(lane == j, total, acc16)

        out_v[pl.ds(g * _L, _L)] = acc16
        return acc16

    lax.fori_loop(0, _G, group, jnp.zeros((_L,), jnp.float32))

    pltpu.sync_copy(out_v, out_hbm.at[pl.ds(base, _BPW)])


def kernel(user_ids, item_ids, user_emb, item_emb, user_bias, item_bias,
           global_bias):
    uid = user_ids.astype(jnp.int32)
    iid = item_ids.astype(jnp.int32)
    return _mf_score(uid, iid, user_emb.T, item_emb.T, user_bias.T,
                     item_bias.T,
                     jnp.broadcast_to(global_bias.reshape(-1)[:1], (_L,)))


# speedup vs baseline: 2.7020x; 1.2174x over previous
import functools

import jax
import jax.numpy as jnp
from jax import lax
from jax.experimental import pallas as pl
from jax.experimental.pallas import tpu as pltpu
from jax.experimental.pallas import tpu_sc as plsc

_B = 16384
_D = 64
_N = 1000000
_NC = 2
_NS = 16
_NW = _NC * _NS
_BPW = _B // _NW
_L = 16
_G = _BPW // _L
_W = 128
_R = 4


@functools.partial(
    pl.kernel,
    mesh=plsc.VectorSubcoreMesh(core_axis_name="c", subcore_axis_name="s"),
    out_type=jax.ShapeDtypeStruct((_B,), jnp.float32),
    compiler_params=pltpu.CompilerParams(
        needs_layout_passes=False, use_tc_tiling_on_sc=True),
    scratch_types=[
        pltpu.VMEM((_BPW,), jnp.int32),
        pltpu.VMEM((_BPW,), jnp.int32),
    ] + [pltpu.VMEM((_D, _W), jnp.float32)] * (2 * _R)
      + [pltpu.VMEM((1, _W), jnp.float32)] * (2 * _R)
      + [
        pltpu.VMEM((_L,), jnp.float32),
        pltpu.VMEM((_BPW,), jnp.float32),
    ] + [pltpu.SemaphoreType.DMA] * _R,
)
def _mf_score(uid_hbm, iid_hbm, uT_hbm, iT_hbm, ubT_hbm, ibT_hbm,
              gb_hbm, out_hbm,
              uid_v, iid_v, *rest):
    ue = rest[0:_R]
    ie = rest[_R:2 * _R]
    ub = rest[2 * _R:3 * _R]
    ib = rest[3 * _R:4 * _R]
    gb_v, out_v = rest[4 * _R], rest[4 * _R + 1]
    sems = rest[4 * _R + 2:]

    wid = lax.axis_index("s") * _NC + lax.axis_index("c")
    base = wid * _BPW

    pltpu.sync_copy(uid_hbm.at[pl.ds(base, _BPW)], uid_v)
    pltpu.sync_copy(iid_hbm.at[pl.ds(base, _BPW)], iid_v)
    pltpu.sync_copy(gb_hbm, gb_v)

    def fire(uscalar, iscalar, p):
        uoff = pl.multiple_of((uscalar >> 7) * _W, _W)
        ioff = pl.multiple_of((iscalar >> 7) * _W, _W)
        pltpu.async_copy(uT_hbm.at[:, pl.ds(uoff, _W)], ue[p], sems[p])
        pltpu.async_copy(iT_hbm.at[:, pl.ds(ioff, _W)], ie[p], sems[p])
        pltpu.async_copy(ubT_hbm.at[:, pl.ds(uoff, _W)], ub[p], sems[p])
        pltpu.async_copy(ibT_hbm.at[:, pl.ds(ioff, _W)], ib[p], sems[p])

    def drain(p):
        pltpu.make_async_copy(uT_hbm.at[:, pl.ds(0, _W)], ue[p], sems[p]).wait()
        pltpu.make_async_copy(iT_hbm.at[:, pl.ds(0, _W)], ie[p], sems[p]).wait()
        pltpu.make_async_copy(ubT_hbm.at[:, pl.ds(0, _W)], ub[p], sems[p]).wait()
        pltpu.make_async_copy(ibT_hbm.at[:, pl.ds(0, _W)], ib[p], sems[p]).wait()

    u16p = uid_v[pl.ds(0, _L)]
    i16p = iid_v[pl.ds(0, _L)]
    for p in range(_R):
        fire(u16p[p], i16p[p], p)

    gb0 = gb_v[pl.ds(0, _L)][0]
    lane = lax.iota(jnp.int32, _L)
    zero16 = jnp.zeros((_L,), jnp.int32)

    def group(g, acc16):
        u16 = uid_v[pl.ds(g * _L, _L)]
        i16 = iid_v[pl.ds(g * _L, _L)]
        gnext = jnp.minimum(g + 1, _G - 1) * _L
        un16 = uid_v[pl.ds(gnext, _L)]
        in16 = iid_v[pl.ds(gnext, _L)]

        for j in range(_L):
            p = j % _R
            drain(p)
            cu = jnp.full((_L,), u16[j] & (_W - 1), jnp.int32)
            ci = jnp.full((_L,), i16[j] & (_W - 1), jnp.int32)
            s = jnp.zeros((_L,), jnp.float32)
            for q in range(_D // _L):
                rows = q * _L + lane
                u = plsc.load_gather(ue[p], [rows, cu])
                it = plsc.load_gather(ie[p], [rows, ci])
                s = s + u * it
            bias = (plsc.load_gather(ub[p], [zero16, cu])
                    + plsc.load_gather(ib[p], [zero16, ci]))

            if j < _L - _R:
                fire(u16[j + _R], i16[j + _R], p)
            else:
                @pl.when(g < _G - 1)
                def _(j=j, p=p):
                    fire(un16[j + _R - _L], in16[j + _R - _L], p)

            total = jnp.sum(s) + bias[0] + gb0
            acc16 = jnp.where(lane == j, total, acc16)

        out_v[pl.ds(g * _L, _L)] = acc16
        return acc16

    lax.fori_loop(0, _G, group, jnp.zeros((_L,), jnp.float32))

    pltpu.sync_copy(out_v, out_hbm.at[pl.ds(base, _BPW)])


def kernel(user_ids, item_ids, user_emb, item_emb, user_bias, item_bias,
           global_bias):
    uid = user_ids.astype(jnp.int32)
    iid = item_ids.astype(jnp.int32)
    return _mf_score(uid, iid, user_emb.T, item_emb.T, user_bias.T,
                     item_bias.T,
                     jnp.broadcast_to(global_bias.reshape(-1)[:1], (_L,)))


# speedup vs baseline: 2.9233x; 1.0819x over previous
import functools

import jax
import jax.numpy as jnp
from jax import lax
from jax.experimental import pallas as pl
from jax.experimental.pallas import tpu as pltpu
from jax.experimental.pallas import tpu_sc as plsc

_B = 16384
_D = 64
_N = 1000000
_NC = 2
_NS = 16
_NW = _NC * _NS
_BPW = _B // _NW
_L = 16
_G = _BPW // _L
_W = 128
_R = 6


@functools.partial(
    pl.kernel,
    mesh=plsc.VectorSubcoreMesh(core_axis_name="c", subcore_axis_name="s"),
    out_type=jax.ShapeDtypeStruct((_B,), jnp.float32),
    compiler_params=pltpu.CompilerParams(
        needs_layout_passes=False, use_tc_tiling_on_sc=True),
    scratch_types=[
        pltpu.VMEM((_BPW,), jnp.int32),
        pltpu.VMEM((_BPW,), jnp.int32),
    ] + [pltpu.VMEM((_D, _W), jnp.float32)] * (2 * _R)
      + [pltpu.VMEM((1, _W), jnp.float32)] * (2 * _R)
      + [
        pltpu.VMEM((_L,), jnp.float32),
        pltpu.VMEM((_BPW,), jnp.float32),
    ] + [pltpu.SemaphoreType.DMA] * _R,
)
def _mf_score(uid_hbm, iid_hbm, uT_hbm, iT_hbm, ubT_hbm, ibT_hbm,
              gb_hbm, out_hbm,
              uid_v, iid_v, *rest):
    ue = rest[0:_R]
    ie = rest[_R:2 * _R]
    ub = rest[2 * _R:3 * _R]
    ib = rest[3 * _R:4 * _R]
    gb_v, out_v = rest[4 * _R], rest[4 * _R + 1]
    sems = rest[4 * _R + 2:]

    wid = lax.axis_index("s") * _NC + lax.axis_index("c")
    base = wid * _BPW

    pltpu.sync_copy(uid_hbm.at[pl.ds(base, _BPW)], uid_v)
    pltpu.sync_copy(iid_hbm.at[pl.ds(base, _BPW)], iid_v)
    pltpu.sync_copy(gb_hbm, gb_v)

    def fire(uscalar, iscalar, p):
        uoff = pl.multiple_of((uscalar >> 7) * _W, _W)
        ioff = pl.multiple_of((iscalar >> 7) * _W, _W)
        pltpu.async_copy(uT_hbm.at[:, pl.ds(uoff, _W)], ue[p], sems[p])
        pltpu.async_copy(iT_hbm.at[:, pl.ds(ioff, _W)], ie[p], sems[p])
        pltpu.async_copy(ubT_hbm.at[:, pl.ds(uoff, _W)], ub[p], sems[p])
        pltpu.async_copy(ibT_hbm.at[:, pl.ds(ioff, _W)], ib[p], sems[p])

    def drain(p):
        pltpu.make_async_copy(uT_hbm.at[:, pl.ds(0, _W)], ue[p], sems[p]).wait()
        pltpu.make_async_copy(iT_hbm.at[:, pl.ds(0, _W)], ie[p], sems[p]).wait()
        pltpu.make_async_copy(ubT_hbm.at[:, pl.ds(0, _W)], ub[p], sems[p]).wait()
        pltpu.make_async_copy(ibT_hbm.at[:, pl.ds(0, _W)], ib[p], sems[p]).wait()

    u16p = uid_v[pl.ds(0, _L)]
    i16p = iid_v[pl.ds(0, _L)]
    for p in range(_R):
        fire(u16p[p], i16p[p], p)

    gb0 = gb_v[pl.ds(0, _L)][0]
    lane = lax.iota(jnp.int32, _L)
    zero16 = jnp.zeros((_L,), jnp.int32)

    def group(g, acc16):
        u16 = uid_v[pl.ds(g * _L, _L)]
        i16 = iid_v[pl.ds(g * _L, _L)]
        gnext = jnp.minimum(g + 1, _G - 1) * _L
        un16 = uid_v[pl.ds(gnext, _L)]
        in16 = iid_v[pl.ds(gnext, _L)]

        for j in range(_L):
            p = j % _R
            drain(p)
            cu = jnp.full((_L,), u16[j] & (_W - 1), jnp.int32)
            ci = jnp.full((_L,), i16[j] & (_W - 1), jnp.int32)
            s = jnp.zeros((_L,), jnp.float32)
            for q in range(_D // _L):
                rows = q * _L + lane
                u = plsc.load_gather(ue[p], [rows, cu])
                it = plsc.load_gather(ie[p], [rows, ci])
                s = s + u * it
            bias = (plsc.load_gather(ub[p], [zero16, cu])
                    + plsc.load_gather(ib[p], [zero16, ci]))

            if j < _L - _R:
                fire(u16[j + _R], i16[j + _R], p)
            else:
                @pl.when(g < _G - 1)
                def _(j=j, p=p):
                    fire(un16[j + _R - _L], in16[j + _R - _L], p)

            total = jnp.sum(s) + bias[0] + gb0
            acc16 = jnp.where(lane == j, total, acc16)

        out_v[pl.ds(g * _L, _L)] = acc16
        return acc16

    lax.fori_loop(0, _G, group, jnp.zeros((_L,), jnp.float32))

    pltpu.sync_copy(out_v, out_hbm.at[pl.ds(base, _BPW)])


def kernel(user_ids, item_ids, user_emb, item_emb, user_bias, item_bias,
           global_bias):
    uid = user_ids.astype(jnp.int32)
    iid = item_ids.astype(jnp.int32)
    return _mf_score(uid, iid, user_emb.T, item_emb.T, user_bias.T,
                     item_bias.T,
                     jnp.broadcast_to(global_bias.reshape(-1)[:1], (_L,)))
